# Initial kernel scaffold; baseline (speedup 1.0000x reference)
import functools

import jax
import jax.numpy as jnp
from jax import lax
from jax.experimental import pallas as pl

N = 10000
E = 320000
D = 128
R = 16
B = 4096

NB = 1000


def _proj_body(emb_ref, w_ref, rel_ref, p_ref, q_ref):
    p = jax.lax.dot_general(
        emb_ref[...], w_ref[0],
        (((1,), (0,)), ((), ())),
        preferred_element_type=jnp.float32,
        precision=lax.Precision.HIGHEST,
    )
    p_ref[0] = p
    q_ref[0] = jnp.tanh(p + rel_ref[0][None, :])


def _project(entity_embed, W_r, relation_embed):
    grid = (R, N // NB)
    return pl.pallas_call(
        _proj_body,
        grid=grid,
        in_specs=[
            pl.BlockSpec((NB, D), lambda r, n: (n, 0)),
            pl.BlockSpec((1, D, D), lambda r, n: (r, 0, 0)),
            pl.BlockSpec((1, D), lambda r, n: (r, 0)),
        ],
        out_specs=[
            pl.BlockSpec((1, NB, D), lambda r, n: (r, n, 0)),
            pl.BlockSpec((1, NB, D), lambda r, n: (r, n, 0)),
        ],
        out_shape=[
            jax.ShapeDtypeStruct((R, N, D), jnp.float32),
            jax.ShapeDtypeStruct((R, N, D), jnp.float32),
        ],
    )(entity_embed, W_r, relation_embed)


def _layer_body(x_ref, hacc_ref, den_ref, w1_ref, b1_ref, w2_ref, b2_ref, o_ref):
    den = jnp.sum(den_ref[...], axis=0)
    rden = 1.0 / (den + 1e-10)
    h = jnp.sum(hacc_ref[...], axis=0) * rden[:, None]
    x = x_ref[...]
    s = x + h
    m = x * h
    y1 = jax.lax.dot_general(
        s, w1_ref[...], (((1,), (0,)), ((), ())),
        preferred_element_type=jnp.float32, precision=lax.Precision.HIGHEST,
    ) + b1_ref[...][None, :]
    y2 = jax.lax.dot_general(
        m, w2_ref[...], (((1,), (0,)), ((), ())),
        preferred_element_type=jnp.float32, precision=lax.Precision.HIGHEST,
    ) + b2_ref[...][None, :]
    o_ref[...] = jnp.where(y1 > 0, y1, 0.01 * y1) + jnp.where(y2 > 0, y2, 0.01 * y2)


def _layer(x, hacc, den_parts, W1, b1, W2, b2):
    S = hacc.shape[0]
    T = den_parts.shape[0]
    grid = (N // NB,)
    return pl.pallas_call(
        _layer_body,
        grid=grid,
        in_specs=[
            pl.BlockSpec((NB, D), lambda n: (n, 0)),
            pl.BlockSpec((S, NB, D), lambda n: (0, n, 0)),
            pl.BlockSpec((T, NB), lambda n: (0, n)),
            pl.BlockSpec((D, D), lambda n: (0, 0)),
            pl.BlockSpec((D,), lambda n: (0,)),
            pl.BlockSpec((D, D), lambda n: (0, 0)),
            pl.BlockSpec((D,), lambda n: (0,)),
        ],
        out_specs=pl.BlockSpec((NB, D), lambda n: (n, 0)),
        out_shape=jax.ShapeDtypeStruct((N, D), jnp.float32),
    )(x, hacc, den_parts, W1, b1, W2, b2)


def kernel(edge_index, edge_type, users, items, entity_embed, relation_embed,
           W_r, W1_0, b1_0, W2_0, b2_0, W1_1, b1_1, W2_1, b2_1):
    src = edge_index[0]
    dst = edge_index[1]

    P, Q = _project(entity_embed, W_r, relation_embed)
    Pf = P.reshape(R * N, D)
    Qf = Q.reshape(R * N, D)

    idx_tail = edge_type * N + src
    idx_head = edge_type * N + dst

    tail = Pf[idx_tail]
    headq = Qf[idx_head]
    a = jnp.sum(tail * headq, axis=-1)
    ex = jnp.exp(a)
    den = jax.ops.segment_sum(ex, dst, num_segments=N)
    den_parts = den[None, :]

    x = entity_embed
    embs = [x]
    for (W1, b1, W2, b2) in ((W1_0, b1_0, W2_0, b2_0), (W1_1, b1_1, W2_1, b2_1)):
        m = ex[:, None] * x[src]
        hacc = jax.ops.segment_sum(m, dst, num_segments=N)[None, :, :]
        x = _layer(x, hacc, den_parts, W1, b1, W2, b2)
        embs.append(x)

    final = jnp.concatenate(embs, axis=1)
    scores = jnp.sum(final[users] * final[items], axis=1)
    return scores


# speedup vs baseline: 2.0996x; 2.0996x over previous
import functools

import jax
import jax.numpy as jnp
from jax import lax
from jax.experimental import pallas as pl

N = 10000
E = 320000
D = 128
R = 16
B = 4096

NB = 1000


def _proj_body(emb_ref, w_ref, rel_ref, p_ref, q_ref):
    p = jax.lax.dot_general(
        emb_ref[...], w_ref[0],
        (((1,), (0,)), ((), ())),
        preferred_element_type=jnp.float32,
        precision=lax.Precision.HIGHEST,
    )
    r = pl.program_id(0)
    p_ref[0] = p
    q_ref[0] = jnp.tanh(p + rel_ref[r][None, :])


def _project(entity_embed, W_r, relation_embed):
    grid = (R, N // NB)
    return pl.pallas_call(
        _proj_body,
        grid=grid,
        in_specs=[
            pl.BlockSpec((NB, D), lambda r, n: (n, 0)),
            pl.BlockSpec((1, D, D), lambda r, n: (r, 0, 0)),
            pl.BlockSpec((R, D), lambda r, n: (0, 0)),
        ],
        out_specs=[
            pl.BlockSpec((1, NB, D), lambda r, n: (r, n, 0)),
            pl.BlockSpec((1, NB, D), lambda r, n: (r, n, 0)),
        ],
        out_shape=[
            jax.ShapeDtypeStruct((R, N, D), jnp.float32),
            jax.ShapeDtypeStruct((R, N, D), jnp.float32),
        ],
    )(entity_embed, W_r, relation_embed)


def _layer_body(x_ref, hacc_ref, den_ref, w1_ref, b1_ref, w2_ref, b2_ref, o_ref):
    den = jnp.sum(den_ref[...], axis=0)
    rden = 1.0 / (den + 1e-10)
    h = jnp.sum(hacc_ref[...], axis=0) * rden
    x = x_ref[...]
    s = x + h
    m = x * h
    y1 = jax.lax.dot_general(
        s, w1_ref[...], (((1,), (0,)), ((), ())),
        preferred_element_type=jnp.float32, precision=lax.Precision.HIGHEST,
    ) + b1_ref[...][None, :]
    y2 = jax.lax.dot_general(
        m, w2_ref[...], (((1,), (0,)), ((), ())),
        preferred_element_type=jnp.float32, precision=lax.Precision.HIGHEST,
    ) + b2_ref[...][None, :]
    o_ref[...] = jnp.where(y1 > 0, y1, 0.01 * y1) + jnp.where(y2 > 0, y2, 0.01 * y2)


def _layer(x, hacc, den_parts, W1, b1, W2, b2):
    S = hacc.shape[0]
    T = den_parts.shape[0]
    grid = (N // NB,)
    return pl.pallas_call(
        _layer_body,
        grid=grid,
        in_specs=[
            pl.BlockSpec((NB, D), lambda n: (n, 0)),
            pl.BlockSpec((S, NB, D), lambda n: (0, n, 0)),
            pl.BlockSpec((T, NB, 1), lambda n: (0, n, 0)),
            pl.BlockSpec((D, D), lambda n: (0, 0)),
            pl.BlockSpec((D,), lambda n: (0,)),
            pl.BlockSpec((D, D), lambda n: (0, 0)),
            pl.BlockSpec((D,), lambda n: (0,)),
        ],
        out_specs=pl.BlockSpec((NB, D), lambda n: (n, 0)),
        out_shape=jax.ShapeDtypeStruct((N, D), jnp.float32),
    )(x, hacc, den_parts, W1, b1, W2, b2)


def kernel(edge_index, edge_type, users, items, entity_embed, relation_embed,
           W_r, W1_0, b1_0, W2_0, b2_0, W1_1, b1_1, W2_1, b2_1):
    src = edge_index[0]
    dst = edge_index[1]

    P, Q = _project(entity_embed, W_r, relation_embed)
    Pf = P.reshape(R * N, D)
    Qf = Q.reshape(R * N, D)

    idx_tail = edge_type * N + src
    idx_head = edge_type * N + dst

    tail = Pf[idx_tail]
    headq = Qf[idx_head]
    a = jnp.sum(tail * headq, axis=-1)
    ex = jnp.exp(a)
    den = jax.ops.segment_sum(ex, dst, num_segments=N)
    den_parts = den[None, :, None]

    x = entity_embed
    embs = [x]
    for (W1, b1, W2, b2) in ((W1_0, b1_0, W2_0, b2_0), (W1_1, b1_1, W2_1, b2_1)):
        m = ex[:, None] * x[src]
        hacc = jax.ops.segment_sum(m, dst, num_segments=N)[None, :, :]
        x = _layer(x, hacc, den_parts, W1, b1, W2, b2)
        embs.append(x)

    final = jnp.concatenate(embs, axis=1)
    scores = jnp.sum(final[users] * final[items], axis=1)
    return scores


# speedup vs baseline: 2.3252x; 1.1075x over previous
import functools

import jax
import jax.numpy as jnp
from jax import lax
from jax.experimental import pallas as pl
from jax.experimental.pallas import tpu as pltpu
from jax.experimental.pallas import tpu_sc as plsc

N = 10000
E = 320000
D = 128
R = 16
B = 4096

NB = 1000

NC = 2
NS = 16
L = 16
NW = NC * NS
CH = 256
NCHUNKS = E // CH
CPW = -(-NCHUNKS // NW)
DSEG = D // L


def _proj_body(emb_ref, w_ref, rel_ref, p_ref, q_ref):
    p = jax.lax.dot_general(
        emb_ref[...], w_ref[0],
        (((1,), (0,)), ((), ())),
        preferred_element_type=jnp.float32,
        precision=lax.Precision.HIGHEST,
    )
    r = pl.program_id(0)
    p_ref[0] = p
    q_ref[0] = jnp.tanh(p + rel_ref[r][None, :])


def _project(entity_embed, W_r, relation_embed):
    grid = (R, N // NB)
    return pl.pallas_call(
        _proj_body,
        grid=grid,
        in_specs=[
            pl.BlockSpec((NB, D), lambda r, n: (n, 0)),
            pl.BlockSpec((1, D, D), lambda r, n: (r, 0, 0)),
            pl.BlockSpec((R, D), lambda r, n: (0, 0)),
        ],
        out_specs=[
            pl.BlockSpec((1, NB, D), lambda r, n: (r, n, 0)),
            pl.BlockSpec((1, NB, D), lambda r, n: (r, n, 0)),
        ],
        out_shape=[
            jax.ShapeDtypeStruct((R, N, D), jnp.float32),
            jax.ShapeDtypeStruct((R, N, D), jnp.float32),
        ],
    )(entity_embed, W_r, relation_embed)


def _layer_body(x_ref, hacc_ref, den_ref, w1_ref, b1_ref, w2_ref, b2_ref, o_ref):
    den = jnp.sum(den_ref[...], axis=0)
    rden = 1.0 / (den + 1e-10)
    h = jnp.sum(hacc_ref[...], axis=0) * rden
    x = x_ref[...]
    s = x + h
    m = x * h
    y1 = jax.lax.dot_general(
        s, w1_ref[...], (((1,), (0,)), ((), ())),
        preferred_element_type=jnp.float32, precision=lax.Precision.HIGHEST,
    ) + b1_ref[...][None, :]
    y2 = jax.lax.dot_general(
        m, w2_ref[...], (((1,), (0,)), ((), ())),
        preferred_element_type=jnp.float32, precision=lax.Precision.HIGHEST,
    ) + b2_ref[...][None, :]
    o_ref[...] = jnp.where(y1 > 0, y1, 0.01 * y1) + jnp.where(y2 > 0, y2, 0.01 * y2)


def _layer(x, hacc, den_parts, W1, b1, W2, b2):
    S = hacc.shape[0]
    T = den_parts.shape[0]
    grid = (N // NB,)
    return pl.pallas_call(
        _layer_body,
        grid=grid,
        in_specs=[
            pl.BlockSpec((NB, D), lambda n: (n, 0)),
            pl.BlockSpec((S, NB, D), lambda n: (0, n, 0)),
            pl.BlockSpec((T, NB, 1), lambda n: (0, n, 0)),
            pl.BlockSpec((D, D), lambda n: (0, 0)),
            pl.BlockSpec((D,), lambda n: (0,)),
            pl.BlockSpec((D, D), lambda n: (0, 0)),
            pl.BlockSpec((D,), lambda n: (0,)),
        ],
        out_specs=pl.BlockSpec((NB, D), lambda n: (n, 0)),
        out_shape=jax.ShapeDtypeStruct((N, D), jnp.float32),
    )(x, hacc, den_parts, W1, b1, W2, b2)


def _attn_body(src_hbm, dst_hbm, typ_hbm, p_hbm, q_hbm, ex_hbm, den_hbm,
               src_v, dst_v, typ_v, tidx, hidx, prow, qrow, accb, exb, den_v,
               sem_p, sem_q):
    wid = lax.axis_index("s") * NC + lax.axis_index("c")
    iota = lax.iota(jnp.int32, L)

    def zden(i, c):
        den_v[pl.ds(i * L, L)] = jnp.zeros((L,), jnp.float32)
        return c
    lax.fori_loop(0, N // L, zden, 0)

    def chunk(k, c):
        cid = wid + k * NW

        @pl.when(cid < NCHUNKS)
        def _():
            base = cid * CH
            pltpu.sync_copy(src_hbm.at[pl.ds(base, CH)], src_v)
            pltpu.sync_copy(dst_hbm.at[pl.ds(base, CH)], dst_v)
            pltpu.sync_copy(typ_hbm.at[pl.ds(base, CH)], typ_v)

            for r in range(CH // 128):
                def idxloop(i, c2, r=r):
                    sl = pl.ds(r * 128 + i * L, L)
                    sl2 = pl.ds(i * L, L)
                    t = typ_v[sl] * N
                    tidx[r, sl2] = t + src_v[sl]
                    hidx[r, sl2] = t + dst_v[sl]
                    return c2
                lax.fori_loop(0, 128 // L, idxloop, 0)

            cps = []
            for r in range(CH // 128):
                cps.append(pltpu.async_copy(
                    p_hbm.at[tidx.at[r]], prow.at[pl.ds(r * 128, 128)], sem_p))
                cps.append(pltpu.async_copy(
                    q_hbm.at[hidx.at[r]], qrow.at[pl.ds(r * 128, 128)], sem_q))
            for cp in cps:
                cp.wait()

            def group(g, c2):
                for kk in range(L):
                    e = g * L + kk
                    acc = prow[e, pl.ds(0, L)] * qrow[e, pl.ds(0, L)]
                    for cc in range(1, DSEG):
                        acc = acc + (prow[e, pl.ds(cc * L, L)]
                                     * qrow[e, pl.ds(cc * L, L)])
                    accb[pl.ds(kk * L, L)] = acc
                res = plsc.load_gather(accb, [iota * L])
                for ll in range(1, L):
                    res = res + plsc.load_gather(accb, [iota * L + ll])
                exv = jnp.exp(res)
                exb[pl.ds(g * L, L)] = exv
                plsc.addupdate_scatter(den_v, [dst_v[pl.ds(g * L, L)]], exv)
                return c2
            lax.fori_loop(0, CH // L, group, 0)
            pltpu.sync_copy(exb, ex_hbm.at[pl.ds(base, CH)])
        return c

    lax.fori_loop(0, CPW, chunk, 0)
    pltpu.sync_copy(den_v, den_hbm.at[wid])


def _attention(src, dst, typ, Pf, Qf):
    mesh = plsc.VectorSubcoreMesh(core_axis_name="c", subcore_axis_name="s")
    f = pl.kernel(
        _attn_body,
        out_type=[
            jax.ShapeDtypeStruct((E,), jnp.float32),
            jax.ShapeDtypeStruct((NW, N), jnp.float32),
        ],
        mesh=mesh,
        scratch_types=[
            pltpu.VMEM((CH,), jnp.int32),
            pltpu.VMEM((CH,), jnp.int32),
            pltpu.VMEM((CH,), jnp.int32),
            pltpu.VMEM((CH // 128, 128), jnp.int32),
            pltpu.VMEM((CH // 128, 128), jnp.int32),
            pltpu.VMEM((CH, D), jnp.float32),
            pltpu.VMEM((CH, D), jnp.float32),
            pltpu.VMEM((L * L,), jnp.float32),
            pltpu.VMEM((CH,), jnp.float32),
            pltpu.VMEM((N,), jnp.float32),
            pltpu.SemaphoreType.DMA,
            pltpu.SemaphoreType.DMA,
        ],
        compiler_params=pltpu.CompilerParams(needs_layout_passes=False),
    )
    return f(src, dst, typ, Pf, Qf)


def kernel(edge_index, edge_type, users, items, entity_embed, relation_embed,
           W_r, W1_0, b1_0, W2_0, b2_0, W1_1, b1_1, W2_1, b2_1):
    src = edge_index[0]
    dst = edge_index[1]

    P, Q = _project(entity_embed, W_r, relation_embed)
    Pf = P.reshape(R * N, D)
    Qf = Q.reshape(R * N, D)

    ex, den_parts2 = _attention(src, dst, edge_type, Pf, Qf)
    den_parts = den_parts2[:, :, None]

    x = entity_embed
    embs = [x]
    for (W1, b1, W2, b2) in ((W1_0, b1_0, W2_0, b2_0), (W1_1, b1_1, W2_1, b2_1)):
        m = ex[:, None] * x[src]
        hacc = jax.ops.segment_sum(m, dst, num_segments=N)[None, :, :]
        x = _layer(x, hacc, den_parts, W1, b1, W2, b2)
        embs.append(x)

    final = jnp.concatenate(embs, axis=1)
    scores = jnp.sum(final[users] * final[items], axis=1)
    return scores


# speedup vs baseline: 6.2367x; 2.6822x over previous
import functools

import jax
import jax.numpy as jnp
from jax import lax
from jax.experimental import pallas as pl
from jax.experimental.pallas import tpu as pltpu
from jax.experimental.pallas import tpu_sc as plsc

N = 10000
E = 320000
D = 128
R = 16
B = 4096

NB = 1000

NC = 2
NS = 16
L = 16
NW = NC * NS
CH = 256
NCHUNKS = E // CH
CPW = -(-NCHUNKS // NW)
DSEG = D // L


def _proj_body(emb_ref, w_ref, rel_ref, p_ref, q_ref):
    p = jax.lax.dot_general(
        emb_ref[...], w_ref[0],
        (((1,), (0,)), ((), ())),
        preferred_element_type=jnp.float32,
        precision=lax.Precision.HIGHEST,
    )
    r = pl.program_id(0)
    p_ref[0] = p
    q_ref[0] = jnp.tanh(p + rel_ref[r][None, :])


def _project(entity_embed, W_r, relation_embed):
    grid = (R, N // NB)
    return pl.pallas_call(
        _proj_body,
        grid=grid,
        in_specs=[
            pl.BlockSpec((NB, D), lambda r, n: (n, 0)),
            pl.BlockSpec((1, D, D), lambda r, n: (r, 0, 0)),
            pl.BlockSpec((R, D), lambda r, n: (0, 0)),
        ],
        out_specs=[
            pl.BlockSpec((1, NB, D), lambda r, n: (r, n, 0)),
            pl.BlockSpec((1, NB, D), lambda r, n: (r, n, 0)),
        ],
        out_shape=[
            jax.ShapeDtypeStruct((R, N, D), jnp.float32),
            jax.ShapeDtypeStruct((R, N, D), jnp.float32),
        ],
    )(entity_embed, W_r, relation_embed)


def _layer_body(x_ref, hacc_ref, den_ref, w1_ref, b1_ref, w2_ref, b2_ref, o_ref):
    den = jnp.sum(den_ref[...], axis=0)
    rden = 1.0 / (den + 1e-10)
    h = jnp.sum(hacc_ref[...], axis=0) * rden
    x = x_ref[...]
    s = x + h
    m = x * h
    y1 = jax.lax.dot_general(
        s, w1_ref[...], (((1,), (0,)), ((), ())),
        preferred_element_type=jnp.float32, precision=lax.Precision.HIGHEST,
    ) + b1_ref[...][None, :]
    y2 = jax.lax.dot_general(
        m, w2_ref[...], (((1,), (0,)), ((), ())),
        preferred_element_type=jnp.float32, precision=lax.Precision.HIGHEST,
    ) + b2_ref[...][None, :]
    o_ref[...] = jnp.where(y1 > 0, y1, 0.01 * y1) + jnp.where(y2 > 0, y2, 0.01 * y2)


def _layer(x, hacc, den_parts, W1, b1, W2, b2):
    S = hacc.shape[0]
    T = den_parts.shape[0]
    grid = (N // NB,)
    return pl.pallas_call(
        _layer_body,
        grid=grid,
        in_specs=[
            pl.BlockSpec((NB, D), lambda n: (n, 0)),
            pl.BlockSpec((S, NB, D), lambda n: (0, n, 0)),
            pl.BlockSpec((T, NB, 1), lambda n: (0, n, 0)),
            pl.BlockSpec((D, D), lambda n: (0, 0)),
            pl.BlockSpec((D,), lambda n: (0,)),
            pl.BlockSpec((D, D), lambda n: (0, 0)),
            pl.BlockSpec((D,), lambda n: (0,)),
        ],
        out_specs=pl.BlockSpec((NB, D), lambda n: (n, 0)),
        out_shape=jax.ShapeDtypeStruct((N, D), jnp.float32),
    )(x, hacc, den_parts, W1, b1, W2, b2)


def _attn_body(src_hbm, dst_hbm, typ_hbm, p_hbm, q_hbm, ex_hbm, den_hbm,
               src_v, dst_v, typ_v, tidx, hidx, prow, qrow, accb, exb, den_v,
               sem_p, sem_q):
    wid = lax.axis_index("s") * NC + lax.axis_index("c")
    iota = lax.iota(jnp.int32, L)

    def zden(i, c):
        den_v[pl.ds(i * L, L)] = jnp.zeros((L,), jnp.float32)
        return c
    lax.fori_loop(0, N // L, zden, 0)

    def chunk(k, c):
        cid = wid + k * NW

        @pl.when(cid < NCHUNKS)
        def _():
            base = cid * CH
            pltpu.sync_copy(src_hbm.at[pl.ds(base, CH)], src_v)
            pltpu.sync_copy(dst_hbm.at[pl.ds(base, CH)], dst_v)
            pltpu.sync_copy(typ_hbm.at[pl.ds(base, CH)], typ_v)

            for r in range(CH // 128):
                def idxloop(i, c2, r=r):
                    sl = pl.ds(r * 128 + i * L, L)
                    sl2 = pl.ds(i * L, L)
                    t = typ_v[sl] * N
                    tidx[r, sl2] = t + src_v[sl]
                    hidx[r, sl2] = t + dst_v[sl]
                    return c2
                lax.fori_loop(0, 128 // L, idxloop, 0)

            cps = []
            for r in range(CH // 128):
                cps.append(pltpu.async_copy(
                    p_hbm.at[tidx.at[r]], prow.at[pl.ds(r * 128, 128)], sem_p))
                cps.append(pltpu.async_copy(
                    q_hbm.at[hidx.at[r]], qrow.at[pl.ds(r * 128, 128)], sem_q))
            for cp in cps:
                cp.wait()

            def group(g, c2):
                for kk in range(L):
                    e = g * L + kk
                    acc = prow[e, pl.ds(0, L)] * qrow[e, pl.ds(0, L)]
                    for cc in range(1, DSEG):
                        acc = acc + (prow[e, pl.ds(cc * L, L)]
                                     * qrow[e, pl.ds(cc * L, L)])
                    accb[pl.ds(kk * L, L)] = acc
                res = plsc.load_gather(accb, [iota * L])
                for ll in range(1, L):
                    res = res + plsc.load_gather(accb, [iota * L + ll])
                exv = jnp.exp(res)
                exb[pl.ds(g * L, L)] = exv
                plsc.addupdate_scatter(den_v, [dst_v[pl.ds(g * L, L)]], exv)
                return c2
            lax.fori_loop(0, CH // L, group, 0)
            pltpu.sync_copy(exb, ex_hbm.at[pl.ds(base, CH)])
        return c

    lax.fori_loop(0, CPW, chunk, 0)
    pltpu.sync_copy(den_v, den_hbm.at[wid])


def _attention(src, dst, typ, Pf, Qf):
    mesh = plsc.VectorSubcoreMesh(core_axis_name="c", subcore_axis_name="s")
    f = pl.kernel(
        _attn_body,
        out_type=[
            jax.ShapeDtypeStruct((E,), jnp.float32),
            jax.ShapeDtypeStruct((NW, N), jnp.float32),
        ],
        mesh=mesh,
        scratch_types=[
            pltpu.VMEM((CH,), jnp.int32),
            pltpu.VMEM((CH,), jnp.int32),
            pltpu.VMEM((CH,), jnp.int32),
            pltpu.VMEM((CH // 128, 128), jnp.int32),
            pltpu.VMEM((CH // 128, 128), jnp.int32),
            pltpu.VMEM((CH, D), jnp.float32),
            pltpu.VMEM((CH, D), jnp.float32),
            pltpu.VMEM((L * L,), jnp.float32),
            pltpu.VMEM((CH,), jnp.float32),
            pltpu.VMEM((N,), jnp.float32),
            pltpu.SemaphoreType.DMA,
            pltpu.SemaphoreType.DMA,
        ],
        compiler_params=pltpu.CompilerParams(needs_layout_passes=False),
    )
    return f(src, dst, typ, Pf, Qf)


NROW = N // NS
def _prop_body(src_hbm, dst_hbm, ex_hbm, x_hbm, hacc_hbm,
               sidx, didx, ex_v, xrow, hsh, sem):
    cid = lax.axis_index("c")
    sid = lax.axis_index("s")
    wid = sid * NC + cid

    def zrow(i, c):
        for cc in range(DSEG):
            xrow[i, pl.ds(cc * L, L)] = jnp.zeros((L,), jnp.float32)
        return c
    lax.fori_loop(0, CH, zrow, 0)
    for t in range(3):
        pltpu.sync_copy(xrow.at[pl.ds(0, 208)],
                        hsh.at[pl.ds(sid * 624 + t * 208, 208)])

    @pl.when(sid == 0)
    def _():
        pltpu.sync_copy(xrow.at[pl.ds(0, 16)], hsh.at[pl.ds(9984, 16)])
    plsc.subcore_barrier()

    def chunk(k, c):
        ck = wid + k * NW

        @pl.when(ck < NCHUNKS)
        def _():
            base = ck * CH
            for r in range(CH // 128):
                pltpu.sync_copy(src_hbm.at[pl.ds(base + r * 128, 128)], sidx.at[r])
                pltpu.sync_copy(dst_hbm.at[pl.ds(base + r * 128, 128)], didx.at[r])
            pltpu.sync_copy(ex_hbm.at[pl.ds(base, CH)], ex_v)
            cps = [pltpu.async_copy(x_hbm.at[sidx.at[r]],
                                    xrow.at[pl.ds(r * 128, 128)], sem)
                   for r in range(CH // 128)]
            for cp in cps:
                cp.wait()

            def group(g, c2):
                exv16 = ex_v[pl.ds(g * L, L)]
                for kk in range(L):
                    e = g * L + kk
                    s = exv16[kk]
                    for cc in range(DSEG):
                        sl = pl.ds(cc * L, L)
                        xrow[e, sl] = xrow[e, sl] * s
                return c2
            lax.fori_loop(0, CH // L, group, 0)
            for r in range(CH // 128):
                pltpu.sync_copy(xrow.at[pl.ds(r * 128, 128)],
                                hsh.at[didx.at[r]], add=True)
        return c

    lax.fori_loop(0, CPW, chunk, 0)
    plsc.subcore_barrier()
    pltpu.sync_copy(hsh.at[pl.ds(sid * 624, 624)],
                    hacc_hbm.at[cid, pl.ds(sid * 624, 624)])

    @pl.when(sid == 0)
    def _():
        pltpu.sync_copy(hsh.at[pl.ds(9984, 16)],
                        hacc_hbm.at[cid, pl.ds(9984, 16)])


def _propagate(src, dst, ex, x):
    mesh = plsc.VectorSubcoreMesh(core_axis_name="c", subcore_axis_name="s")
    f = pl.kernel(
        _prop_body,
        out_type=jax.ShapeDtypeStruct((NC, N, D), jnp.float32),
        mesh=mesh,
        scratch_types=[
            pltpu.VMEM((CH // 128, 128), jnp.int32),
            pltpu.VMEM((CH // 128, 128), jnp.int32),
            pltpu.VMEM((CH,), jnp.float32),
            pltpu.VMEM((CH, D), jnp.float32),
            pltpu.VMEM_SHARED((N, D), jnp.float32),
            pltpu.SemaphoreType.DMA,
        ],
        compiler_params=pltpu.CompilerParams(needs_layout_passes=False),
    )
    return f(src, dst, ex, x)


def kernel(edge_index, edge_type, users, items, entity_embed, relation_embed,
           W_r, W1_0, b1_0, W2_0, b2_0, W1_1, b1_1, W2_1, b2_1):
    src = edge_index[0]
    dst = edge_index[1]

    P, Q = _project(entity_embed, W_r, relation_embed)
    Pf = P.reshape(R * N, D)
    Qf = Q.reshape(R * N, D)

    ex, den_parts2 = _attention(src, dst, edge_type, Pf, Qf)
    den_parts = den_parts2[:, :, None]

    x = entity_embed
    embs = [x]
    for (W1, b1, W2, b2) in ((W1_0, b1_0, W2_0, b2_0), (W1_1, b1_1, W2_1, b2_1)):
        hacc = _propagate(src, dst, ex, x)
        x = _layer(x, hacc, den_parts, W1, b1, W2, b2)
        embs.append(x)

    final = jnp.concatenate(embs, axis=1)
    scores = jnp.sum(final[users] * final[items], axis=1)
    return scores


# speedup vs baseline: 6.3255x; 1.0142x over previous
import functools

import jax
import jax.numpy as jnp
from jax import lax
from jax.experimental import pallas as pl
from jax.experimental.pallas import tpu as pltpu
from jax.experimental.pallas import tpu_sc as plsc

N = 10000
E = 320000
D = 128
R = 16
B = 4096

NB = 1000

NC = 2
NS = 16
L = 16
NW = NC * NS
CH = 256
NCHUNKS = E // CH
CPW = -(-NCHUNKS // NW)
DSEG = D // L


def _proj_body(emb_ref, w_ref, rel_ref, p_ref, q_ref):
    p = jax.lax.dot_general(
        emb_ref[...], w_ref[0],
        (((1,), (0,)), ((), ())),
        preferred_element_type=jnp.float32,
        precision=lax.Precision.HIGHEST,
    )
    r = pl.program_id(0)
    p_ref[0] = p
    q_ref[0] = jnp.tanh(p + rel_ref[r][None, :])


def _project(entity_embed, W_r, relation_embed):
    grid = (R, N // NB)
    return pl.pallas_call(
        _proj_body,
        grid=grid,
        in_specs=[
            pl.BlockSpec((NB, D), lambda r, n: (n, 0)),
            pl.BlockSpec((1, D, D), lambda r, n: (r, 0, 0)),
            pl.BlockSpec((R, D), lambda r, n: (0, 0)),
        ],
        out_specs=[
            pl.BlockSpec((1, NB, D), lambda r, n: (r, n, 0)),
            pl.BlockSpec((1, NB, D), lambda r, n: (r, n, 0)),
        ],
        out_shape=[
            jax.ShapeDtypeStruct((R, N, D), jnp.float32),
            jax.ShapeDtypeStruct((R, N, D), jnp.float32),
        ],
    )(entity_embed, W_r, relation_embed)


def _layer_body(x_ref, hacc_ref, den_ref, w1_ref, b1_ref, w2_ref, b2_ref, o_ref):
    den = jnp.sum(den_ref[...], axis=0)
    rden = 1.0 / (den + 1e-10)
    h = jnp.sum(hacc_ref[...], axis=0) * rden
    x = x_ref[...]
    s = x + h
    m = x * h
    y1 = jax.lax.dot_general(
        s, w1_ref[...], (((1,), (0,)), ((), ())),
        preferred_element_type=jnp.float32, precision=lax.Precision.HIGHEST,
    ) + b1_ref[...][None, :]
    y2 = jax.lax.dot_general(
        m, w2_ref[...], (((1,), (0,)), ((), ())),
        preferred_element_type=jnp.float32, precision=lax.Precision.HIGHEST,
    ) + b2_ref[...][None, :]
    o_ref[...] = jnp.where(y1 > 0, y1, 0.01 * y1) + jnp.where(y2 > 0, y2, 0.01 * y2)


def _layer(x, hacc, den_parts, W1, b1, W2, b2):
    S = hacc.shape[0]
    T = den_parts.shape[0]
    grid = (N // NB,)
    return pl.pallas_call(
        _layer_body,
        grid=grid,
        in_specs=[
            pl.BlockSpec((NB, D), lambda n: (n, 0)),
            pl.BlockSpec((S, NB, D), lambda n: (0, n, 0)),
            pl.BlockSpec((T, NB, 1), lambda n: (0, n, 0)),
            pl.BlockSpec((D, D), lambda n: (0, 0)),
            pl.BlockSpec((D,), lambda n: (0,)),
            pl.BlockSpec((D, D), lambda n: (0, 0)),
            pl.BlockSpec((D,), lambda n: (0,)),
        ],
        out_specs=pl.BlockSpec((NB, D), lambda n: (n, 0)),
        out_shape=jax.ShapeDtypeStruct((N, D), jnp.float32),
    )(x, hacc, den_parts, W1, b1, W2, b2)


def _attn_body(src_hbm, dst_hbm, typ_hbm, p_hbm, q_hbm, ex_hbm, den_hbm,
               src_v, dst_v, typ_v, tidx, hidx, prow, qrow, accb, exb, den_v,
               sem_p, sem_q):
    wid = lax.axis_index("s") * NC + lax.axis_index("c")
    iota = lax.iota(jnp.int32, L)

    def zden(i, c):
        den_v[pl.ds(i * L, L)] = jnp.zeros((L,), jnp.float32)
        return c
    lax.fori_loop(0, N // L, zden, 0)

    def chunk(k, c):
        cid = wid + k * NW

        @pl.when(cid < NCHUNKS)
        def _():
            base = cid * CH
            pltpu.sync_copy(src_hbm.at[pl.ds(base, CH)], src_v)
            pltpu.sync_copy(dst_hbm.at[pl.ds(base, CH)], dst_v)
            pltpu.sync_copy(typ_hbm.at[pl.ds(base, CH)], typ_v)

            for r in range(CH // 128):
                def idxloop(i, c2, r=r):
                    sl = pl.ds(r * 128 + i * L, L)
                    sl2 = pl.ds(i * L, L)
                    t = typ_v[sl] * N
                    tidx[r, sl2] = t + src_v[sl]
                    hidx[r, sl2] = t + dst_v[sl]
                    return c2
                lax.fori_loop(0, 128 // L, idxloop, 0)

            cps = []
            for r in range(CH // 128):
                cps.append(pltpu.async_copy(
                    p_hbm.at[tidx.at[r]], prow.at[pl.ds(r * 128, 128)], sem_p))
                cps.append(pltpu.async_copy(
                    q_hbm.at[hidx.at[r]], qrow.at[pl.ds(r * 128, 128)], sem_q))
            for cp in cps:
                cp.wait()

            def group(g, c2):
                for kk in range(L):
                    e = g * L + kk
                    acc = prow[e, pl.ds(0, L)] * qrow[e, pl.ds(0, L)]
                    for cc in range(1, DSEG):
                        acc = acc + (prow[e, pl.ds(cc * L, L)]
                                     * qrow[e, pl.ds(cc * L, L)])
                    accb[pl.ds(kk * L, L)] = acc
                res = plsc.load_gather(accb, [iota * L])
                for ll in range(1, L):
                    res = res + plsc.load_gather(accb, [iota * L + ll])
                exv = jnp.exp(res)
                exb[pl.ds(g * L, L)] = exv
                plsc.addupdate_scatter(den_v, [dst_v[pl.ds(g * L, L)]], exv)
                return c2
            lax.fori_loop(0, CH // L, group, 0)
            pltpu.sync_copy(exb, ex_hbm.at[pl.ds(base, CH)])
        return c

    lax.fori_loop(0, CPW, chunk, 0)
    pltpu.sync_copy(den_v, den_hbm.at[wid])


def _attention(src, dst, typ, Pf, Qf):
    mesh = plsc.VectorSubcoreMesh(core_axis_name="c", subcore_axis_name="s")
    f = pl.kernel(
        _attn_body,
        out_type=[
            jax.ShapeDtypeStruct((E,), jnp.float32),
            jax.ShapeDtypeStruct((NW, N), jnp.float32),
        ],
        mesh=mesh,
        scratch_types=[
            pltpu.VMEM((CH,), jnp.int32),
            pltpu.VMEM((CH,), jnp.int32),
            pltpu.VMEM((CH,), jnp.int32),
            pltpu.VMEM((CH // 128, 128), jnp.int32),
            pltpu.VMEM((CH // 128, 128), jnp.int32),
            pltpu.VMEM((CH, D), jnp.float32),
            pltpu.VMEM((CH, D), jnp.float32),
            pltpu.VMEM((L * L,), jnp.float32),
            pltpu.VMEM((CH,), jnp.float32),
            pltpu.VMEM((N,), jnp.float32),
            pltpu.SemaphoreType.DMA,
            pltpu.SemaphoreType.DMA,
        ],
        compiler_params=pltpu.CompilerParams(needs_layout_passes=False),
    )
    return f(src, dst, typ, Pf, Qf)


NROW = N // NS
def _prop_body(src_hbm, dst_hbm, ex_hbm, x_hbm, hacc_hbm,
               sidx, didx, ex_v, xrow, hsh, sem):
    cid = lax.axis_index("c")
    sid = lax.axis_index("s")
    wid = sid * NC + cid

    def zrow(i, c):
        for cc in range(DSEG):
            xrow[i, pl.ds(cc * L, L)] = jnp.zeros((L,), jnp.float32)
        return c
    lax.fori_loop(0, CH, zrow, 0)
    for t in range(3):
        pltpu.sync_copy(xrow.at[pl.ds(0, 208)],
                        hsh.at[pl.ds(sid * 624 + t * 208, 208)])

    @pl.when(sid == 0)
    def _():
        pltpu.sync_copy(xrow.at[pl.ds(0, 16)], hsh.at[pl.ds(9984, 16)])
    plsc.subcore_barrier()

    def chunk(k, c):
        ck = wid + k * NW

        @pl.when(ck < NCHUNKS)
        def _():
            base = ck * CH
            for r in range(CH // 128):
                pltpu.sync_copy(src_hbm.at[pl.ds(base + r * 128, 128)], sidx.at[r])
                pltpu.sync_copy(dst_hbm.at[pl.ds(base + r * 128, 128)], didx.at[r])
            pltpu.sync_copy(ex_hbm.at[pl.ds(base, CH)], ex_v)
            cps = [pltpu.async_copy(x_hbm.at[sidx.at[r]],
                                    xrow.at[pl.ds(r * 128, 128)], sem)
                   for r in range(CH // 128)]
            for cp in cps:
                cp.wait()

            def group(g, c2):
                exv16 = ex_v[pl.ds(g * L, L)]
                for kk in range(L):
                    e = g * L + kk
                    s = exv16[kk]
                    for cc in range(DSEG):
                        sl = pl.ds(cc * L, L)
                        xrow[e, sl] = xrow[e, sl] * s
                return c2
            lax.fori_loop(0, CH // L, group, 0)
            for r in range(CH // 128):
                pltpu.sync_copy(xrow.at[pl.ds(r * 128, 128)],
                                hsh.at[didx.at[r]], add=True)
        return c

    lax.fori_loop(0, CPW, chunk, 0)
    plsc.subcore_barrier()
    pltpu.sync_copy(hsh.at[pl.ds(sid * 624, 624)],
                    hacc_hbm.at[cid, pl.ds(sid * 624, 624)])

    @pl.when(sid == 0)
    def _():
        pltpu.sync_copy(hsh.at[pl.ds(9984, 16)],
                        hacc_hbm.at[cid, pl.ds(9984, 16)])


def _propagate(src, dst, ex, x):
    mesh = plsc.VectorSubcoreMesh(core_axis_name="c", subcore_axis_name="s")
    f = pl.kernel(
        _prop_body,
        out_type=jax.ShapeDtypeStruct((NC, N, D), jnp.float32),
        mesh=mesh,
        scratch_types=[
            pltpu.VMEM((CH // 128, 128), jnp.int32),
            pltpu.VMEM((CH // 128, 128), jnp.int32),
            pltpu.VMEM((CH,), jnp.float32),
            pltpu.VMEM((CH, D), jnp.float32),
            pltpu.VMEM_SHARED((N, D), jnp.float32),
            pltpu.SemaphoreType.DMA,
        ],
        compiler_params=pltpu.CompilerParams(needs_layout_passes=False),
    )
    return f(src, dst, ex, x)


BPW = B // NW
def _score_body(u_hbm, i_hbm, x0_hbm, x1_hbm, x2_hbm, out_hbm,
                uidx, iidx, urow, irow, accb, sbuf, sem_u, sem_i):
    wid = lax.axis_index("s") * NC + lax.axis_index("c")
    iota = lax.iota(jnp.int32, L)
    base = wid * BPW
    pltpu.sync_copy(u_hbm.at[pl.ds(base, BPW)], uidx.at[0])
    pltpu.sync_copy(i_hbm.at[pl.ds(base, BPW)], iidx.at[0])
    for g in range(BPW // L):
        sbuf[pl.ds(g * L, L)] = jnp.zeros((L,), jnp.float32)
    for x_hbm in (x0_hbm, x1_hbm, x2_hbm):
        cu = pltpu.async_copy(x_hbm.at[uidx.at[0]], urow, sem_u)
        ci = pltpu.async_copy(x_hbm.at[iidx.at[0]], irow, sem_i)
        cu.wait()
        ci.wait()

        def group(g, c):
            for kk in range(L):
                e = g * L + kk
                acc = urow[e, pl.ds(0, L)] * irow[e, pl.ds(0, L)]
                for cc in range(1, DSEG):
                    acc = acc + (urow[e, pl.ds(cc * L, L)]
                                 * irow[e, pl.ds(cc * L, L)])
                accb[pl.ds(kk * L, L)] = acc
            res = plsc.load_gather(accb, [iota * L])
            for ll in range(1, L):
                res = res + plsc.load_gather(accb, [iota * L + ll])
            sl = pl.ds(g * L, L)
            sbuf[sl] = sbuf[sl] + res
            return c
        lax.fori_loop(0, BPW // L, group, 0)
    pltpu.sync_copy(sbuf, out_hbm.at[pl.ds(base, BPW)])


def _score(users, items, x0, x1, x2):
    mesh = plsc.VectorSubcoreMesh(core_axis_name="c", subcore_axis_name="s")
    f = pl.kernel(
        _score_body,
        out_type=jax.ShapeDtypeStruct((B,), jnp.float32),
        mesh=mesh,
        scratch_types=[
            pltpu.VMEM((1, BPW), jnp.int32),
            pltpu.VMEM((1, BPW), jnp.int32),
            pltpu.VMEM((BPW, D), jnp.float32),
            pltpu.VMEM((BPW, D), jnp.float32),
            pltpu.VMEM((L * L,), jnp.float32),
            pltpu.VMEM((BPW,), jnp.float32),
            pltpu.SemaphoreType.DMA,
            pltpu.SemaphoreType.DMA,
        ],
        compiler_params=pltpu.CompilerParams(needs_layout_passes=False),
    )
    return f(users, items, x0, x1, x2)


def kernel(edge_index, edge_type, users, items, entity_embed, relation_embed,
           W_r, W1_0, b1_0, W2_0, b2_0, W1_1, b1_1, W2_1, b2_1):
    src = edge_index[0]
    dst = edge_index[1]

    P, Q = _project(entity_embed, W_r, relation_embed)
    Pf = P.reshape(R * N, D)
    Qf = Q.reshape(R * N, D)

    ex, den_parts2 = _attention(src, dst, edge_type, Pf, Qf)
    den_parts = den_parts2[:, :, None]

    x = entity_embed
    embs = [x]
    for (W1, b1, W2, b2) in ((W1_0, b1_0, W2_0, b2_0), (W1_1, b1_1, W2_1, b2_1)):
        hacc = _propagate(src, dst, ex, x)
        x = _layer(x, hacc, den_parts, W1, b1, W2, b2)
        embs.append(x)

    scores = _score(users, items, embs[0], embs[1], embs[2])
    return scores


# speedup vs baseline: 7.3600x; 1.1636x over previous
import functools

import jax
import jax.numpy as jnp
from jax import lax
from jax.experimental import pallas as pl
from jax.experimental.pallas import tpu as pltpu
from jax.experimental.pallas import tpu_sc as plsc

N = 10000
E = 320000
D = 128
R = 16
B = 4096

NB = 1000

NC = 2
NS = 16
L = 16
NW = NC * NS
CH = 256
NCHUNKS = E // CH
CPW = -(-NCHUNKS // NW)
DSEG = D // L


def _proj_body(emb_ref, w_ref, rel_ref, p_ref, q_ref):
    p = jax.lax.dot_general(
        emb_ref[...], w_ref[0],
        (((1,), (0,)), ((), ())),
        preferred_element_type=jnp.float32,
        precision=lax.Precision.HIGHEST,
    )
    r = pl.program_id(0)
    p_ref[0] = p
    q_ref[0] = jnp.tanh(p + rel_ref[r][None, :])


def _project(entity_embed, W_r, relation_embed):
    grid = (R, N // NB)
    return pl.pallas_call(
        _proj_body,
        grid=grid,
        in_specs=[
            pl.BlockSpec((NB, D), lambda r, n: (n, 0)),
            pl.BlockSpec((1, D, D), lambda r, n: (r, 0, 0)),
            pl.BlockSpec((R, D), lambda r, n: (0, 0)),
        ],
        out_specs=[
            pl.BlockSpec((1, NB, D), lambda r, n: (r, n, 0)),
            pl.BlockSpec((1, NB, D), lambda r, n: (r, n, 0)),
        ],
        out_shape=[
            jax.ShapeDtypeStruct((R, N, D), jnp.float32),
            jax.ShapeDtypeStruct((R, N, D), jnp.float32),
        ],
    )(entity_embed, W_r, relation_embed)


def _layer_body(x_ref, hacc_ref, den_ref, w1_ref, b1_ref, w2_ref, b2_ref, o_ref):
    den = jnp.sum(den_ref[...], axis=0)
    rden = 1.0 / (den + 1e-10)
    h = jnp.sum(hacc_ref[...], axis=0) * rden
    x = x_ref[...]
    s = x + h
    m = x * h
    y1 = jax.lax.dot_general(
        s, w1_ref[...], (((1,), (0,)), ((), ())),
        preferred_element_type=jnp.float32, precision=lax.Precision.HIGHEST,
    ) + b1_ref[...][None, :]
    y2 = jax.lax.dot_general(
        m, w2_ref[...], (((1,), (0,)), ((), ())),
        preferred_element_type=jnp.float32, precision=lax.Precision.HIGHEST,
    ) + b2_ref[...][None, :]
    o_ref[...] = jnp.where(y1 > 0, y1, 0.01 * y1) + jnp.where(y2 > 0, y2, 0.01 * y2)


def _layer(x, hacc, den_parts, W1, b1, W2, b2):
    S = hacc.shape[0]
    T = den_parts.shape[0]
    grid = (N // NB,)
    return pl.pallas_call(
        _layer_body,
        grid=grid,
        in_specs=[
            pl.BlockSpec((NB, D), lambda n: (n, 0)),
            pl.BlockSpec((S, NB, D), lambda n: (0, n, 0)),
            pl.BlockSpec((T, NB, 1), lambda n: (0, n, 0)),
            pl.BlockSpec((D, D), lambda n: (0, 0)),
            pl.BlockSpec((D,), lambda n: (0,)),
            pl.BlockSpec((D, D), lambda n: (0, 0)),
            pl.BlockSpec((D,), lambda n: (0,)),
        ],
        out_specs=pl.BlockSpec((NB, D), lambda n: (n, 0)),
        out_shape=jax.ShapeDtypeStruct((N, D), jnp.float32),
    )(x, hacc, den_parts, W1, b1, W2, b2)


CE = 80
KC = E // NW // CE
GPC = CE // L


def _attn_body(src_hbm, dst_hbm, typ_hbm, p_hbm, q_hbm, ex_hbm, den_hbm,
               src_a, dst_a, typ_a, tixA, hixA, tixB, hixB,
               pA, qA, pB, qB, accb, ex_a, den_v,
               semPA, semQA, semPB, semQB):
    wid = lax.axis_index("s") * NC + lax.axis_index("c")
    iota = lax.iota(jnp.int32, L)

    def zden(i, c):
        den_v[pl.ds(i * L, L)] = jnp.zeros((L,), jnp.float32)
        return c
    lax.fori_loop(0, N // L, zden, 0)

    e0 = wid * (KC * CE)
    pltpu.sync_copy(src_hbm.at[pl.ds(e0, KC * CE)], src_a)
    pltpu.sync_copy(dst_hbm.at[pl.ds(e0, KC * CE)], dst_a)
    pltpu.sync_copy(typ_hbm.at[pl.ds(e0, KC * CE)], typ_a)

    def fire(k, tix, hix, pbuf, qbuf, semP, semQ):
        def bg(i, c):
            sl = pl.ds(k * CE + i * L, L)
            t = typ_a[sl] * N
            tix[0, pl.ds(i * L, L)] = t + src_a[sl]
            hix[0, pl.ds(i * L, L)] = t + dst_a[sl]
            return c
        lax.fori_loop(0, GPC, bg, 0)
        pltpu.async_copy(p_hbm.at[tix.at[0]], pbuf, semP)
        pltpu.async_copy(q_hbm.at[hix.at[0]], qbuf, semQ)

    def waitg(tix, hix, pbuf, qbuf, semP, semQ):
        pltpu.make_async_copy(p_hbm.at[tix.at[0]], pbuf, semP).wait()
        pltpu.make_async_copy(q_hbm.at[hix.at[0]], qbuf, semQ).wait()

    def compute(k, pbuf, qbuf):
        def grp(g, c):
            for kk in range(L):
                e = g * L + kk
                acc = pbuf[e, pl.ds(0, L)] * qbuf[e, pl.ds(0, L)]
                for cc in range(1, DSEG):
                    acc = acc + (pbuf[e, pl.ds(cc * L, L)]
                                 * qbuf[e, pl.ds(cc * L, L)])
                accb[pl.ds(kk * L, L)] = acc
            res = plsc.load_gather(accb, [iota * L])
            for ll in range(1, L):
                res = res + plsc.load_gather(accb, [iota * L + ll])
            exv = jnp.exp(res)
            sl = pl.ds(k * CE + g * L, L)
            ex_a[sl] = exv
            plsc.addupdate_scatter(den_v, [dst_a[sl]], exv)
            return c
        lax.fori_loop(0, GPC, grp, 0)

    fire(0, tixA, hixA, pA, qA, semPA, semQA)
    fire(1, tixB, hixB, pB, qB, semPB, semQB)

    def pair(k2, c):
        a = 2 * k2
        waitg(tixA, hixA, pA, qA, semPA, semQA)
        fire(a + 2, tixA, hixA, pA, qA, semPA, semQA)
        compute(a, pA, qA)
        waitg(tixB, hixB, pB, qB, semPB, semQB)

        @pl.when(a + 3 < KC)
        def _():
            fire(a + 3, tixB, hixB, pB, qB, semPB, semQB)
        compute(a + 1, pB, qB)
        return c
    lax.fori_loop(0, (KC - 1) // 2, pair, 0)
    waitg(tixA, hixA, pA, qA, semPA, semQA)
    compute(KC - 1, pA, qA)

    pltpu.sync_copy(ex_a, ex_hbm.at[pl.ds(e0, KC * CE)])
    pltpu.sync_copy(den_v, den_hbm.at[wid])


def _attention(src, dst, typ, Pf, Qf):
    mesh = plsc.VectorSubcoreMesh(core_axis_name="c", subcore_axis_name="s")
    f = pl.kernel(
        _attn_body,
        out_type=[
            jax.ShapeDtypeStruct((E,), jnp.float32),
            jax.ShapeDtypeStruct((NW, N), jnp.float32),
        ],
        mesh=mesh,
        scratch_types=[
            pltpu.VMEM((KC * CE,), jnp.int32),
            pltpu.VMEM((KC * CE,), jnp.int32),
            pltpu.VMEM((KC * CE,), jnp.int32),
            pltpu.VMEM((1, CE), jnp.int32),
            pltpu.VMEM((1, CE), jnp.int32),
            pltpu.VMEM((1, CE), jnp.int32),
            pltpu.VMEM((1, CE), jnp.int32),
            pltpu.VMEM((CE, D), jnp.float32),
            pltpu.VMEM((CE, D), jnp.float32),
            pltpu.VMEM((CE, D), jnp.float32),
            pltpu.VMEM((CE, D), jnp.float32),
            pltpu.VMEM((L * L,), jnp.float32),
            pltpu.VMEM((KC * CE,), jnp.float32),
            pltpu.VMEM((N,), jnp.float32),
            pltpu.SemaphoreType.DMA,
            pltpu.SemaphoreType.DMA,
            pltpu.SemaphoreType.DMA,
            pltpu.SemaphoreType.DMA,
        ],
        compiler_params=pltpu.CompilerParams(needs_layout_passes=False),
    )
    return f(src, dst, typ, Pf, Qf)


NROW = N // NS
def _prop_body(src_hbm, dst_hbm, ex_hbm, x_hbm, hacc_hbm,
               sidx, didx, ex_v, xrow, hsh, sem):
    cid = lax.axis_index("c")
    sid = lax.axis_index("s")
    wid = sid * NC + cid

    def zrow(i, c):
        for cc in range(DSEG):
            xrow[i, pl.ds(cc * L, L)] = jnp.zeros((L,), jnp.float32)
        return c
    lax.fori_loop(0, CH, zrow, 0)
    for t in range(3):
        pltpu.sync_copy(xrow.at[pl.ds(0, 208)],
                        hsh.at[pl.ds(sid * 624 + t * 208, 208)])

    @pl.when(sid == 0)
    def _():
        pltpu.sync_copy(xrow.at[pl.ds(0, 16)], hsh.at[pl.ds(9984, 16)])
    plsc.subcore_barrier()

    def chunk(k, c):
        ck = wid + k * NW

        @pl.when(ck < NCHUNKS)
        def _():
            base = ck * CH
            for r in range(CH // 128):
                pltpu.sync_copy(src_hbm.at[pl.ds(base + r * 128, 128)], sidx.at[r])
                pltpu.sync_copy(dst_hbm.at[pl.ds(base + r * 128, 128)], didx.at[r])
            pltpu.sync_copy(ex_hbm.at[pl.ds(base, CH)], ex_v)
            cps = [pltpu.async_copy(x_hbm.at[sidx.at[r]],
                                    xrow.at[pl.ds(r * 128, 128)], sem)
                   for r in range(CH // 128)]
            for cp in cps:
                cp.wait()

            def group(g, c2):
                exv16 = ex_v[pl.ds(g * L, L)]
                for kk in range(L):
                    e = g * L + kk
                    s = exv16[kk]
                    for cc in range(DSEG):
                        sl = pl.ds(cc * L, L)
                        xrow[e, sl] = xrow[e, sl] * s
                return c2
            lax.fori_loop(0, CH // L, group, 0)
            for r in range(CH // 128):
                pltpu.sync_copy(xrow.at[pl.ds(r * 128, 128)],
                                hsh.at[didx.at[r]], add=True)
        return c

    lax.fori_loop(0, CPW, chunk, 0)
    plsc.subcore_barrier()
    pltpu.sync_copy(hsh.at[pl.ds(sid * 624, 624)],
                    hacc_hbm.at[cid, pl.ds(sid * 624, 624)])

    @pl.when(sid == 0)
    def _():
        pltpu.sync_copy(hsh.at[pl.ds(9984, 16)],
                        hacc_hbm.at[cid, pl.ds(9984, 16)])


def _propagate(src, dst, ex, x):
    mesh = plsc.VectorSubcoreMesh(core_axis_name="c", subcore_axis_name="s")
    f = pl.kernel(
        _prop_body,
        out_type=jax.ShapeDtypeStruct((NC, N, D), jnp.float32),
        mesh=mesh,
        scratch_types=[
            pltpu.VMEM((CH // 128, 128), jnp.int32),
            pltpu.VMEM((CH // 128, 128), jnp.int32),
            pltpu.VMEM((CH,), jnp.float32),
            pltpu.VMEM((CH, D), jnp.float32),
            pltpu.VMEM_SHARED((N, D), jnp.float32),
            pltpu.SemaphoreType.DMA,
        ],
        compiler_params=pltpu.CompilerParams(needs_layout_passes=False),
    )
    return f(src, dst, ex, x)


BPW = B // NW
def _score_body(u_hbm, i_hbm, x0_hbm, x1_hbm, x2_hbm, out_hbm,
                uidx, iidx, urow, irow, accb, sbuf, sem_u, sem_i):
    wid = lax.axis_index("s") * NC + lax.axis_index("c")
    iota = lax.iota(jnp.int32, L)
    base = wid * BPW
    pltpu.sync_copy(u_hbm.at[pl.ds(base, BPW)], uidx.at[0])
    pltpu.sync_copy(i_hbm.at[pl.ds(base, BPW)], iidx.at[0])
    for g in range(BPW // L):
        sbuf[pl.ds(g * L, L)] = jnp.zeros((L,), jnp.float32)
    for x_hbm in (x0_hbm, x1_hbm, x2_hbm):
        cu = pltpu.async_copy(x_hbm.at[uidx.at[0]], urow, sem_u)
        ci = pltpu.async_copy(x_hbm.at[iidx.at[0]], irow, sem_i)
        cu.wait()
        ci.wait()

        def group(g, c):
            for kk in range(L):
                e = g * L + kk
                acc = urow[e, pl.ds(0, L)] * irow[e, pl.ds(0, L)]
                for cc in range(1, DSEG):
                    acc = acc + (urow[e, pl.ds(cc * L, L)]
                                 * irow[e, pl.ds(cc * L, L)])
                accb[pl.ds(kk * L, L)] = acc
            res = plsc.load_gather(accb, [iota * L])
            for ll in range(1, L):
                res = res + plsc.load_gather(accb, [iota * L + ll])
            sl = pl.ds(g * L, L)
            sbuf[sl] = sbuf[sl] + res
            return c
        lax.fori_loop(0, BPW // L, group, 0)
    pltpu.sync_copy(sbuf, out_hbm.at[pl.ds(base, BPW)])


def _score(users, items, x0, x1, x2):
    mesh = plsc.VectorSubcoreMesh(core_axis_name="c", subcore_axis_name="s")
    f = pl.kernel(
        _score_body,
        out_type=jax.ShapeDtypeStruct((B,), jnp.float32),
        mesh=mesh,
        scratch_types=[
            pltpu.VMEM((1, BPW), jnp.int32),
            pltpu.VMEM((1, BPW), jnp.int32),
            pltpu.VMEM((BPW, D), jnp.float32),
            pltpu.VMEM((BPW, D), jnp.float32),
            pltpu.VMEM((L * L,), jnp.float32),
            pltpu.VMEM((BPW,), jnp.float32),
            pltpu.SemaphoreType.DMA,
            pltpu.SemaphoreType.DMA,
        ],
        compiler_params=pltpu.CompilerParams(needs_layout_passes=False),
    )
    return f(users, items, x0, x1, x2)


def kernel(edge_index, edge_type, users, items, entity_embed, relation_embed,
           W_r, W1_0, b1_0, W2_0, b2_0, W1_1, b1_1, W2_1, b2_1):
    src = edge_index[0]
    dst = edge_index[1]

    P, Q = _project(entity_embed, W_r, relation_embed)
    Pf = P.reshape(R * N, D)
    Qf = Q.reshape(R * N, D)

    ex, den_parts2 = _attention(src, dst, edge_type, Pf, Qf)
    den_parts = den_parts2[:, :, None]

    x = entity_embed
    embs = [x]
    for (W1, b1, W2, b2) in ((W1_0, b1_0, W2_0, b2_0), (W1_1, b1_1, W2_1, b2_1)):
        hacc = _propagate(src, dst, ex, x)
        x = _layer(x, hacc, den_parts, W1, b1, W2, b2)
        embs.append(x)

    scores = _score(users, items, embs[0], embs[1], embs[2])
    return scores


# speedup vs baseline: 10.2772x; 1.3963x over previous
import functools

import jax
import jax.numpy as jnp
from jax import lax
from jax.experimental import pallas as pl
from jax.experimental.pallas import tpu as pltpu
from jax.experimental.pallas import tpu_sc as plsc

N = 10000
E = 320000
D = 128
R = 16
B = 4096

NB = 1000

NC = 2
NS = 16
L = 16
NW = NC * NS
CH = 256
NCHUNKS = E // CH
CPW = -(-NCHUNKS // NW)
DSEG = D // L


def _proj_body(emb_ref, w_ref, rel_ref, p_ref, q_ref):
    p = jax.lax.dot_general(
        emb_ref[...], w_ref[0],
        (((1,), (0,)), ((), ())),
        preferred_element_type=jnp.float32,
        precision=lax.Precision.HIGHEST,
    )
    r = pl.program_id(0)
    p_ref[0] = p
    q_ref[0] = jnp.tanh(p + rel_ref[r][None, :])


def _project(entity_embed, W_r, relation_embed):
    grid = (R, N // NB)
    return pl.pallas_call(
        _proj_body,
        grid=grid,
        in_specs=[
            pl.BlockSpec((NB, D), lambda r, n: (n, 0)),
            pl.BlockSpec((1, D, D), lambda r, n: (r, 0, 0)),
            pl.BlockSpec((R, D), lambda r, n: (0, 0)),
        ],
        out_specs=[
            pl.BlockSpec((1, NB, D), lambda r, n: (r, n, 0)),
            pl.BlockSpec((1, NB, D), lambda r, n: (r, n, 0)),
        ],
        out_shape=[
            jax.ShapeDtypeStruct((R, N, D), jnp.float32),
            jax.ShapeDtypeStruct((R, N, D), jnp.float32),
        ],
    )(entity_embed, W_r, relation_embed)


def _layer_body(x_ref, hacc_ref, den_ref, w1_ref, b1_ref, w2_ref, b2_ref, o_ref):
    den = jnp.sum(den_ref[...], axis=0)
    rden = 1.0 / (den + 1e-10)
    h = jnp.sum(hacc_ref[...], axis=0) * rden
    x = x_ref[...]
    s = x + h
    m = x * h
    y1 = jax.lax.dot_general(
        s, w1_ref[...], (((1,), (0,)), ((), ())),
        preferred_element_type=jnp.float32, precision=lax.Precision.HIGHEST,
    ) + b1_ref[...][None, :]
    y2 = jax.lax.dot_general(
        m, w2_ref[...], (((1,), (0,)), ((), ())),
        preferred_element_type=jnp.float32, precision=lax.Precision.HIGHEST,
    ) + b2_ref[...][None, :]
    o_ref[...] = jnp.where(y1 > 0, y1, 0.01 * y1) + jnp.where(y2 > 0, y2, 0.01 * y2)


def _layer(x, hacc, den_parts, W1, b1, W2, b2):
    S = hacc.shape[0]
    T = den_parts.shape[0]
    grid = (N // NB,)
    return pl.pallas_call(
        _layer_body,
        grid=grid,
        in_specs=[
            pl.BlockSpec((NB, D), lambda n: (n, 0)),
            pl.BlockSpec((S, NB, D), lambda n: (0, n, 0)),
            pl.BlockSpec((T, NB, 1), lambda n: (0, n, 0)),
            pl.BlockSpec((D, D), lambda n: (0, 0)),
            pl.BlockSpec((D,), lambda n: (0,)),
            pl.BlockSpec((D, D), lambda n: (0, 0)),
            pl.BlockSpec((D,), lambda n: (0,)),
        ],
        out_specs=pl.BlockSpec((NB, D), lambda n: (n, 0)),
        out_shape=jax.ShapeDtypeStruct((N, D), jnp.float32),
    )(x, hacc, den_parts, W1, b1, W2, b2)


CE = 80
KC = E // NW // CE
GPC = CE // L


def _attn_body(src_hbm, dst_hbm, typ_hbm, p_hbm, q_hbm, ex_hbm, den_hbm,
               src_a, dst_a, typ_a, tixA, hixA, tixB, hixB,
               pA, qA, pB, qB, accb, ex_a, den_v,
               semPA, semQA, semPB, semQB):
    wid = lax.axis_index("s") * NC + lax.axis_index("c")
    iota = lax.iota(jnp.int32, L)

    def zden(i, c):
        den_v[pl.ds(i * L, L)] = jnp.zeros((L,), jnp.float32)
        return c
    lax.fori_loop(0, N // L, zden, 0)

    e0 = wid * (KC * CE)
    pltpu.sync_copy(src_hbm.at[pl.ds(e0, KC * CE)], src_a)
    pltpu.sync_copy(dst_hbm.at[pl.ds(e0, KC * CE)], dst_a)
    pltpu.sync_copy(typ_hbm.at[pl.ds(e0, KC * CE)], typ_a)

    def fire(k, tix, hix, pbuf, qbuf, semP, semQ):
        def bg(i, c):
            sl = pl.ds(k * CE + i * L, L)
            t = typ_a[sl] * N
            tix[0, pl.ds(i * L, L)] = t + src_a[sl]
            hix[0, pl.ds(i * L, L)] = t + dst_a[sl]
            return c
        lax.fori_loop(0, GPC, bg, 0)
        pltpu.async_copy(p_hbm.at[tix.at[0]], pbuf, semP)
        pltpu.async_copy(q_hbm.at[hix.at[0]], qbuf, semQ)

    def waitg(tix, hix, pbuf, qbuf, semP, semQ):
        pltpu.make_async_copy(p_hbm.at[tix.at[0]], pbuf, semP).wait()
        pltpu.make_async_copy(q_hbm.at[hix.at[0]], qbuf, semQ).wait()

    def compute(k, pbuf, qbuf):
        def grp(g, c):
            for kk in range(L):
                e = g * L + kk
                acc = pbuf[e, pl.ds(0, L)] * qbuf[e, pl.ds(0, L)]
                for cc in range(1, DSEG):
                    acc = acc + (pbuf[e, pl.ds(cc * L, L)]
                                 * qbuf[e, pl.ds(cc * L, L)])
                accb[pl.ds(kk * L, L)] = acc
            res = plsc.load_gather(accb, [iota * L])
            for ll in range(1, L):
                res = res + plsc.load_gather(accb, [iota * L + ll])
            exv = jnp.exp(res)
            sl = pl.ds(k * CE + g * L, L)
            ex_a[sl] = exv
            plsc.addupdate_scatter(den_v, [dst_a[sl]], exv)
            return c
        lax.fori_loop(0, GPC, grp, 0)

    fire(0, tixA, hixA, pA, qA, semPA, semQA)
    fire(1, tixB, hixB, pB, qB, semPB, semQB)

    def pair(k2, c):
        a = 2 * k2
        waitg(tixA, hixA, pA, qA, semPA, semQA)
        fire(a + 2, tixA, hixA, pA, qA, semPA, semQA)
        compute(a, pA, qA)
        waitg(tixB, hixB, pB, qB, semPB, semQB)

        @pl.when(a + 3 < KC)
        def _():
            fire(a + 3, tixB, hixB, pB, qB, semPB, semQB)
        compute(a + 1, pB, qB)
        return c
    lax.fori_loop(0, (KC - 1) // 2, pair, 0)
    waitg(tixA, hixA, pA, qA, semPA, semQA)
    compute(KC - 1, pA, qA)

    pltpu.sync_copy(ex_a, ex_hbm.at[pl.ds(e0, KC * CE)])
    pltpu.sync_copy(den_v, den_hbm.at[wid])


def _attention(src, dst, typ, Pf, Qf):
    mesh = plsc.VectorSubcoreMesh(core_axis_name="c", subcore_axis_name="s")
    f = pl.kernel(
        _attn_body,
        out_type=[
            jax.ShapeDtypeStruct((E,), jnp.float32),
            jax.ShapeDtypeStruct((NW, N), jnp.float32),
        ],
        mesh=mesh,
        scratch_types=[
            pltpu.VMEM((KC * CE,), jnp.int32),
            pltpu.VMEM((KC * CE,), jnp.int32),
            pltpu.VMEM((KC * CE,), jnp.int32),
            pltpu.VMEM((1, CE), jnp.int32),
            pltpu.VMEM((1, CE), jnp.int32),
            pltpu.VMEM((1, CE), jnp.int32),
            pltpu.VMEM((1, CE), jnp.int32),
            pltpu.VMEM((CE, D), jnp.float32),
            pltpu.VMEM((CE, D), jnp.float32),
            pltpu.VMEM((CE, D), jnp.float32),
            pltpu.VMEM((CE, D), jnp.float32),
            pltpu.VMEM((L * L,), jnp.float32),
            pltpu.VMEM((KC * CE,), jnp.float32),
            pltpu.VMEM((N,), jnp.float32),
            pltpu.SemaphoreType.DMA,
            pltpu.SemaphoreType.DMA,
            pltpu.SemaphoreType.DMA,
            pltpu.SemaphoreType.DMA,
        ],
        compiler_params=pltpu.CompilerParams(needs_layout_passes=False),
    )
    return f(src, dst, typ, Pf, Qf)


def _prop_body(src_hbm, dst_hbm, ex_hbm, x_hbm, hacc_hbm,
               gix0, gix1, gix2, gix3, six0, six1, six2, six3,
               exb0, exb1, exb2, exb3,
               xb0, xb1, xb2, xb3, hsh,
               sm0, sm1, sm2, sm3, sg0, sg1, sg2, sg3, ss0, ss1, ss2, ss3):
    cid = lax.axis_index("c")
    sid = lax.axis_index("s")
    wid = sid * NC + cid
    gix = (gix0, gix1, gix2, gix3)
    six = (six0, six1, six2, six3)
    exb = (exb0, exb1, exb2, exb3)
    xb = (xb0, xb1, xb2, xb3)
    sm = (sm0, sm1, sm2, sm3)
    sg = (sg0, sg1, sg2, sg3)
    ss = (ss0, ss1, ss2, ss3)

    def zrow(i, c):
        for cc in range(DSEG):
            xb0[i, pl.ds(cc * L, L)] = jnp.zeros((L,), jnp.float32)
        return c
    lax.fori_loop(0, CE, zrow, 0)
    for t in range(7):
        pltpu.sync_copy(xb0, hsh.at[pl.ds(sid * 624 + t * 80, 80)])
    pltpu.sync_copy(xb0.at[pl.ds(0, 64)], hsh.at[pl.ds(sid * 624 + 560, 64)])

    @pl.when(sid == 0)
    def _():
        pltpu.sync_copy(xb0.at[pl.ds(0, 16)], hsh.at[pl.ds(9984, 16)])

    e0 = wid * (KC * CE)
    plsc.subcore_barrier()

    def fire_small(k, b):
        base = e0 + k * CE
        pltpu.async_copy(src_hbm.at[pl.ds(base, CE)], gix[b].at[0], sm[b])
        pltpu.async_copy(dst_hbm.at[pl.ds(base, CE)], six[b].at[0], sm[b])
        pltpu.async_copy(ex_hbm.at[pl.ds(base, CE)], exb[b].at[0], sm[b])

    def wait_small(k, b):
        base = e0 + k * CE
        pltpu.make_async_copy(src_hbm.at[pl.ds(base, CE)], gix[b].at[0], sm[b]).wait()
        pltpu.make_async_copy(dst_hbm.at[pl.ds(base, CE)], six[b].at[0], sm[b]).wait()
        pltpu.make_async_copy(ex_hbm.at[pl.ds(base, CE)], exb[b].at[0], sm[b]).wait()

    def fire_row(b):
        pltpu.async_copy(x_hbm.at[gix[b].at[0]], xb[b], sg[b])

    def wait_row(b):
        pltpu.make_async_copy(x_hbm.at[gix[b].at[0]], xb[b], sg[b]).wait()

    def fire_sc(b):
        pltpu.async_copy(xb[b], hsh.at[six[b].at[0]], ss[b], add=True)

    def wait_sc(b):
        pltpu.make_async_copy(xb[b], hsh.at[six[b].at[0]], ss[b]).wait()

    def compute(b):
        xbb = xb[b]
        exbb = exb[b]

        def grp(g, c):
            exv16 = exbb[0, pl.ds(g * L, L)]
            for kk in range(L):
                e = g * L + kk
                s = exv16[kk]
                for cc in range(DSEG):
                    sl = pl.ds(cc * L, L)
                    xbb[e, sl] = xbb[e, sl] * s
            return c
        lax.fori_loop(0, GPC, grp, 0)

    def step(c, b):
        b1 = (b + 1) % 4
        b2 = (b + 2) % 4

        @pl.when(c + 1 < KC)
        def _():
            wait_small(c + 1, b1)
            fire_row(b1)

        @pl.when(c >= 2)
        def _():
            wait_sc(b2)

        @pl.when(c + 2 < KC)
        def _():
            fire_small(c + 2, b2)
        wait_row(b)
        compute(b)
        fire_sc(b)

    fire_small(0, 0)
    fire_small(1, 1)
    wait_small(0, 0)
    fire_row(0)

    def quad(k4, c):
        for j in range(4):
            step(4 * k4 + j, j)
        return c
    lax.fori_loop(0, KC // 4, quad, 0)
    step(jnp.int32(KC - 1), (KC - 1) % 4)
    wait_sc((KC - 2) % 4)
    wait_sc((KC - 1) % 4)
    plsc.subcore_barrier()

    pltpu.sync_copy(hsh.at[pl.ds(sid * 624, 624)],
                    hacc_hbm.at[cid, pl.ds(sid * 624, 624)])

    @pl.when(sid == 0)
    def _():
        pltpu.sync_copy(hsh.at[pl.ds(9984, 16)],
                        hacc_hbm.at[cid, pl.ds(9984, 16)])


def _propagate(src, dst, ex, x):
    mesh = plsc.VectorSubcoreMesh(core_axis_name="c", subcore_axis_name="s")
    f = pl.kernel(
        _prop_body,
        out_type=jax.ShapeDtypeStruct((NC, N, D), jnp.float32),
        mesh=mesh,
        scratch_types=(
            [pltpu.VMEM((1, CE), jnp.int32) for _ in range(8)]
            + [pltpu.VMEM((1, CE), jnp.float32) for _ in range(4)]
            + [pltpu.VMEM((CE, D), jnp.float32) for _ in range(4)]
            + [pltpu.VMEM_SHARED((N, D), jnp.float32)]
            + [pltpu.SemaphoreType.DMA for _ in range(12)]
        ),
        compiler_params=pltpu.CompilerParams(needs_layout_passes=False),
    )
    return f(src, dst, ex, x)


BPW = B // NW
def _score_body(u_hbm, i_hbm, x0_hbm, x1_hbm, x2_hbm, out_hbm,
                uidx, iidx, urow, irow, accb, sbuf, sem_u, sem_i):
    wid = lax.axis_index("s") * NC + lax.axis_index("c")
    iota = lax.iota(jnp.int32, L)
    base = wid * BPW
    pltpu.sync_copy(u_hbm.at[pl.ds(base, BPW)], uidx.at[0])
    pltpu.sync_copy(i_hbm.at[pl.ds(base, BPW)], iidx.at[0])
    for g in range(BPW // L):
        sbuf[pl.ds(g * L, L)] = jnp.zeros((L,), jnp.float32)
    for x_hbm in (x0_hbm, x1_hbm, x2_hbm):
        cu = pltpu.async_copy(x_hbm.at[uidx.at[0]], urow, sem_u)
        ci = pltpu.async_copy(x_hbm.at[iidx.at[0]], irow, sem_i)
        cu.wait()
        ci.wait()

        def group(g, c):
            for kk in range(L):
                e = g * L + kk
                acc = urow[e, pl.ds(0, L)] * irow[e, pl.ds(0, L)]
                for cc in range(1, DSEG):
                    acc = acc + (urow[e, pl.ds(cc * L, L)]
                                 * irow[e, pl.ds(cc * L, L)])
                accb[pl.ds(kk * L, L)] = acc
            res = plsc.load_gather(accb, [iota * L])
            for ll in range(1, L):
                res = res + plsc.load_gather(accb, [iota * L + ll])
            sl = pl.ds(g * L, L)
            sbuf[sl] = sbuf[sl] + res
            return c
        lax.fori_loop(0, BPW // L, group, 0)
    pltpu.sync_copy(sbuf, out_hbm.at[pl.ds(base, BPW)])


def _score(users, items, x0, x1, x2):
    mesh = plsc.VectorSubcoreMesh(core_axis_name="c", subcore_axis_name="s")
    f = pl.kernel(
        _score_body,
        out_type=jax.ShapeDtypeStruct((B,), jnp.float32),
        mesh=mesh,
        scratch_types=[
            pltpu.VMEM((1, BPW), jnp.int32),
            pltpu.VMEM((1, BPW), jnp.int32),
            pltpu.VMEM((BPW, D), jnp.float32),
            pltpu.VMEM((BPW, D), jnp.float32),
            pltpu.VMEM((L * L,), jnp.float32),
            pltpu.VMEM((BPW,), jnp.float32),
            pltpu.SemaphoreType.DMA,
            pltpu.SemaphoreType.DMA,
        ],
        compiler_params=pltpu.CompilerParams(needs_layout_passes=False),
    )
    return f(users, items, x0, x1, x2)


def kernel(edge_index, edge_type, users, items, entity_embed, relation_embed,
           W_r, W1_0, b1_0, W2_0, b2_0, W1_1, b1_1, W2_1, b2_1):
    src = edge_index[0]
    dst = edge_index[1]

    P, Q = _project(entity_embed, W_r, relation_embed)
    Pf = P.reshape(R * N, D)
    Qf = Q.reshape(R * N, D)

    ex, den_parts2 = _attention(src, dst, edge_type, Pf, Qf)
    den_parts = den_parts2[:, :, None]

    x = entity_embed
    embs = [x]
    for (W1, b1, W2, b2) in ((W1_0, b1_0, W2_0, b2_0), (W1_1, b1_1, W2_1, b2_1)):
        hacc = _propagate(src, dst, ex, x)
        x = _layer(x, hacc, den_parts, W1, b1, W2, b2)
        embs.append(x)

    scores = _score(users, items, embs[0], embs[1], embs[2])
    return scores
